# baseline (speedup 1.0000x reference)
import jax
import jax.numpy as jnp
import numpy as np
from jax import lax
from jax.experimental import pallas as pl
from jax.experimental.pallas import tpu as pltpu
from jax.experimental.pallas import tpu_sc as plsc

N = 16777216
K = 838861
NC, NS = 2, 16
NW = NC * NS
PER_W = N // NW
CHUNK = 16384
NCHUNK = PER_W // CHUNK
UNROLL = 8
VECS = CHUNK // 16

_SIGN = np.int32(-2147483648)
_NEG1 = np.int32(-1)


def _hist_slot(x):
    xi = lax.bitcast_convert_type(x, jnp.int32)
    key = xi ^ jnp.where(xi < 0, _NEG1, _SIGN)
    hi = lax.shift_right_logical(key, jnp.int32(16))
    lo = jnp.bitwise_and(key, jnp.int32(0xFFFF))
    return hi, lo


def _pass1_body(pnl, zeros, hist_out, hist_v, buf0, buf1, sem0, sem1):
    wid = lax.axis_index("s") * NC + lax.axis_index("c")
    base = wid * PER_W
    pltpu.sync_copy(zeros, hist_v)
    bufs = (buf0, buf1)
    sems = (sem0, sem1)
    ones = jnp.ones((16,), jnp.int32)

    def copy(c, b):
        return pltpu.make_async_copy(
            pnl.at[pl.ds(base + c * CHUNK, CHUNK)], bufs[b], sems[b])

    copy(0, 0).start()

    def process(buf, carry):
        def ibody(i):
            x = buf[pl.ds(i * 16, 16)]
            hi, _ = _hist_slot(x)
            plsc.addupdate_scatter(hist_v, [hi], ones)
        plsc.parallel_loop(0, VECS, unroll=UNROLL)(ibody)
        return carry

    def outer(g, carry):
        for b in range(2):
            c = g * 2 + b
            @pl.when(c + 1 < NCHUNK)
            def _():
                copy(c + 1, 1 - b).start()
            copy(c, b).wait()
            carry = process(bufs[b], carry)
        return carry

    lax.fori_loop(0, NCHUNK // 2, outer, jnp.int32(0))
    pltpu.sync_copy(hist_v, hist_out.at[pl.ds(wid * 65536, 65536)])


def _pass2_body(pnl, par, zeros, hist_out,
                hist_v, buf0, buf1, bvec_v, sem0, sem1):
    wid = lax.axis_index("s") * NC + lax.axis_index("c")
    base = wid * PER_W
    pltpu.sync_copy(zeros, hist_v)
    pltpu.sync_copy(par.at[0], bvec_v)
    bv = bvec_v[...]
    bufs = (buf0, buf1)
    sems = (sem0, sem1)
    ones = jnp.ones((16,), jnp.int32)

    def copy(c, b):
        return pltpu.make_async_copy(
            pnl.at[pl.ds(base + c * CHUNK, CHUNK)], bufs[b], sems[b])

    copy(0, 0).start()

    def process(buf, carry):
        def ibody(i):
            x = buf[pl.ds(i * 16, 16)]
            hi, lo = _hist_slot(x)
            plsc.addupdate_scatter(hist_v, [lo], ones, mask=hi == bv)
        plsc.parallel_loop(0, VECS, unroll=UNROLL)(ibody)
        return carry

    def outer(g, carry):
        for b in range(2):
            c = g * 2 + b
            @pl.when(c + 1 < NCHUNK)
            def _():
                copy(c + 1, 1 - b).start()
            copy(c, b).wait()
            carry = process(bufs[b], carry)
        return carry

    lax.fori_loop(0, NCHUNK // 2, outer, jnp.int32(0))
    pltpu.sync_copy(hist_v, hist_out.at[pl.ds(wid * 65536, 65536)])


def _cumlt(C):
    rs = jnp.sum(C, axis=1, keepdims=True)
    ii = lax.broadcasted_iota(jnp.int32, (512, 512), 0)
    jj = lax.broadcasted_iota(jnp.int32, (512, 512), 1)
    lower = (jj < ii).astype(jnp.float32)
    carry = lax.dot(lower, rs, precision=lax.Precision.HIGHEST,
                    preferred_element_type=jnp.float32)
    mm = lax.broadcasted_iota(jnp.int32, (128, 128), 0)
    ll = lax.broadcasted_iota(jnp.int32, (128, 128), 1)
    upper = (mm < ll).astype(jnp.float32)
    within = lax.dot(C, upper, precision=lax.Precision.HIGHEST,
                     preferred_element_type=jnp.float32)
    return carry + within


def _tile_sum(h):
    C = h[0:512]
    for t in range(1, NW):
        C = C + h[t * 512:(t + 1) * 512]
    return C


_FLAT = None


def _flat_idx():
    return (lax.broadcasted_iota(jnp.int32, (512, 128), 0) * 128
            + lax.broadcasted_iota(jnp.int32, (512, 128), 1))


def _merge1_body(hist_ref, out_ref):
    C = _tile_sum(hist_ref[...].astype(jnp.float32))
    cumlt = _cumlt(C)
    cum = cumlt + C
    kf = jnp.float32(K)
    mx = (cumlt < kf) & (kf <= cum)
    flat = _flat_idx()
    b_bin = jnp.sum(jnp.where(mx, flat, 0))
    c_below = jnp.sum(jnp.where(mx, cumlt, 0.0)).astype(jnp.int32)
    key_b = lax.shift_left(b_bin, jnp.int32(16))
    vlo_bits = jnp.where(key_b < 0,
                         jnp.bitwise_and(key_b, jnp.int32(0x7FFFFFFF)),
                         jnp.bitwise_not(key_b))
    rows = lax.broadcasted_iota(jnp.int32, (4, 16), 0)
    out_ref[...] = jnp.where(rows == 0, b_bin,
                             jnp.where(rows == 1, c_below, vlo_bits))


def _sbelow_body(par_ref, x_ref, out_ref):
    pi = pl.program_id(0)

    @pl.when(pi == 0)
    def _():
        out_ref[...] = jnp.zeros((1, 1), jnp.float32)

    vlo = lax.bitcast_convert_type(par_ref[2:3, 0:1], jnp.float32)
    x = x_ref[...]
    s = jnp.sum(jnp.where(x < vlo, x, 0.0))
    out_ref[...] += jnp.broadcast_to(s, (1, 1))


def _merge2_body(hist_ref, sv_ref, par_ref, out_ref):
    C2 = _tile_sum(hist_ref[...].astype(jnp.float32))
    cumlt2 = _cumlt(C2)
    cum2 = cumlt2 + C2
    par = par_ref[...]
    b_bin = jnp.sum(par[0:1, 0:1])
    c_below = jnp.sum(par[1:2, 0:1]).astype(jnp.float32)
    r = jnp.float32(K) - c_below
    flat = _flat_idx()
    key = jnp.bitwise_or(lax.shift_left(b_bin, jnp.int32(16)), flat)
    u = jnp.where(key < 0, jnp.bitwise_and(key, jnp.int32(0x7FFFFFFF)),
                  jnp.bitwise_not(key))
    v = lax.bitcast_convert_type(u, jnp.float32)
    mx = (cumlt2 < r) & (r <= cum2)
    b2 = jnp.sum(jnp.where(mx, flat, 0))
    c_at = jnp.sum(jnp.where(mx, cumlt2, 0.0))
    v_thr = jnp.sum(jnp.where(mx, v, 0.0))
    s_in_bin = jnp.sum(jnp.where(flat < b2, C2 * v, 0.0))
    s_below = jnp.sum(sv_ref[...])
    total = s_below + s_in_bin + (r - c_at) * v_thr
    es = -(total / jnp.float32(K))
    out_ref[...] = jnp.broadcast_to(es, (1, 1))


_mesh = plsc.VectorSubcoreMesh(core_axis_name="c", subcore_axis_name="s")

_sc_params = pltpu.CompilerParams(needs_layout_passes=False)

_pass1 = pl.kernel(
    _pass1_body,
    out_type=jax.ShapeDtypeStruct((NW * 65536,), jnp.int32),
    mesh=_mesh,
    compiler_params=_sc_params,
    scratch_types=[
        pltpu.VMEM((65536,), jnp.int32),
        pltpu.VMEM((CHUNK,), jnp.float32),
        pltpu.VMEM((CHUNK,), jnp.float32),
        pltpu.SemaphoreType.DMA,
        pltpu.SemaphoreType.DMA,
    ],
)

_pass2 = pl.kernel(
    _pass2_body,
    out_type=jax.ShapeDtypeStruct((NW * 65536,), jnp.int32),
    mesh=_mesh,
    compiler_params=_sc_params,
    scratch_types=[
        pltpu.VMEM((65536,), jnp.int32),
        pltpu.VMEM((CHUNK,), jnp.float32),
        pltpu.VMEM((CHUNK,), jnp.float32),
        pltpu.VMEM((16,), jnp.int32),
        pltpu.SemaphoreType.DMA,
        pltpu.SemaphoreType.DMA,
    ],
)

_GRID_SB = 8

_sbelow = pl.pallas_call(
    _sbelow_body,
    grid=(_GRID_SB,),
    in_specs=[
        pl.BlockSpec((4, 16), lambda i: (0, 0)),
        pl.BlockSpec((4096 // _GRID_SB, 4096), lambda i: (i, 0)),
    ],
    out_specs=pl.BlockSpec((1, 1), lambda i: (0, 0)),
    out_shape=jax.ShapeDtypeStruct((1, 1), jnp.float32),
)

_merge1 = pl.pallas_call(
    _merge1_body,
    out_shape=jax.ShapeDtypeStruct((4, 16), jnp.int32),
)

_merge2 = pl.pallas_call(
    _merge2_body,
    out_shape=jax.ShapeDtypeStruct((1, 1), jnp.float32),
)


@jax.jit
def kernel(pnl):
    zeros = jnp.zeros((65536,), jnp.int32)
    hist1 = _pass1(pnl, zeros)
    par = _merge1(hist1.reshape(NW * 512, 128))
    hist2 = _pass2(pnl, par, zeros)
    sb = _sbelow(par, pnl.reshape(4096, 4096))
    out = _merge2(hist2.reshape(NW * 512, 128), sb, par)
    return out[0, 0]


# speedup vs baseline: 1.2216x; 1.2216x over previous
import jax
import jax.numpy as jnp
import numpy as np
from jax import lax
from jax.experimental import pallas as pl
from jax.experimental.pallas import tpu as pltpu
from jax.experimental.pallas import tpu_sc as plsc

N = 16777216
K = 838861
NC, NS = 2, 16
NW = NC * NS
PER_W = N // NW
CHUNK = 16384
NCHUNK = PER_W // CHUNK
UNROLL = 8
VECS = CHUNK // 16

_SIGN = np.int32(-2147483648)
_NEG1 = np.int32(-1)


def _hist_slot(x):
    xi = lax.bitcast_convert_type(x, jnp.int32)
    key = xi ^ jnp.where(xi < 0, _NEG1, _SIGN)
    hi = lax.shift_right_logical(key, jnp.int32(16))
    lo = jnp.bitwise_and(key, jnp.int32(0xFFFF))
    return hi, lo


def _pass1_body(pnl, zeros, hist_out, hist_v, buf0, buf1, sem0, sem1):
    wid = lax.axis_index("s") * NC + lax.axis_index("c")
    base = wid * PER_W
    pltpu.sync_copy(zeros, hist_v)
    bufs = (buf0, buf1)
    sems = (sem0, sem1)
    ones = jnp.ones((16,), jnp.int32)

    def copy(c, b):
        return pltpu.make_async_copy(
            pnl.at[pl.ds(base + c * CHUNK, CHUNK)], bufs[b], sems[b])

    copy(0, 0).start()

    def process(buf, carry):
        def ibody(i):
            x = buf[pl.ds(i * 16, 16)]
            hi, _ = _hist_slot(x)
            plsc.addupdate_scatter(hist_v, [hi], ones)
        plsc.parallel_loop(0, VECS, unroll=UNROLL)(ibody)
        return carry

    def outer(g, carry):
        for b in range(2):
            c = g * 2 + b
            @pl.when(c + 1 < NCHUNK)
            def _():
                copy(c + 1, 1 - b).start()
            copy(c, b).wait()
            carry = process(bufs[b], carry)
        return carry

    lax.fori_loop(0, NCHUNK // 2, outer, jnp.int32(0))
    pltpu.sync_copy(hist_v, hist_out.at[pl.ds(wid * 65536, 65536)])


def _pass2_body(pnl, par, zeros, hist_out, sv_out,
                hist_v, buf0, buf1, bvec_v, accv, sem0, sem1):
    wid = lax.axis_index("s") * NC + lax.axis_index("c")
    base = wid * PER_W
    pltpu.sync_copy(zeros, hist_v)
    pltpu.sync_copy(par.at[0], bvec_v)
    bv = bvec_v[...]
    bufs = (buf0, buf1)
    sems = (sem0, sem1)
    ones = jnp.ones((16,), jnp.int32)
    fzero = jnp.zeros((16,), jnp.float32)

    def copy(c, b):
        return pltpu.make_async_copy(
            pnl.at[pl.ds(base + c * CHUNK, CHUNK)], bufs[b], sems[b])

    copy(0, 0).start()

    def process(buf, acc):
        def ibody(i, acc):
            x = buf[pl.ds(i * 16, 16)]
            hi, lo = _hist_slot(x)
            acc = acc + jnp.where(hi < bv, x, fzero)
            plsc.addupdate_scatter(hist_v, [lo], ones, mask=hi == bv)
            return acc
        return plsc.parallel_loop(0, VECS, unroll=UNROLL, carry=acc)(ibody)

    def outer(g, acc):
        for b in range(2):
            c = g * 2 + b
            @pl.when(c + 1 < NCHUNK)
            def _():
                copy(c + 1, 1 - b).start()
            copy(c, b).wait()
            acc = process(bufs[b], acc)
        return acc

    acc = lax.fori_loop(0, NCHUNK // 2, outer, fzero)
    accv[...] = acc
    pltpu.sync_copy(accv, sv_out.at[wid])
    pltpu.sync_copy(hist_v, hist_out.at[pl.ds(wid * 65536, 65536)])


def _cumlt(C):
    rs = jnp.sum(C, axis=1, keepdims=True)
    ii = lax.broadcasted_iota(jnp.int32, (512, 512), 0)
    jj = lax.broadcasted_iota(jnp.int32, (512, 512), 1)
    lower = (jj < ii).astype(jnp.float32)
    carry = lax.dot(lower, rs, precision=lax.Precision.HIGHEST,
                    preferred_element_type=jnp.float32)
    mm = lax.broadcasted_iota(jnp.int32, (128, 128), 0)
    ll = lax.broadcasted_iota(jnp.int32, (128, 128), 1)
    upper = (mm < ll).astype(jnp.float32)
    within = lax.dot(C, upper, precision=lax.Precision.HIGHEST,
                     preferred_element_type=jnp.float32)
    return carry + within


def _tile_sum(h):
    C = h[0:512]
    for t in range(1, NW):
        C = C + h[t * 512:(t + 1) * 512]
    return C


_FLAT = None


def _flat_idx():
    return (lax.broadcasted_iota(jnp.int32, (512, 128), 0) * 128
            + lax.broadcasted_iota(jnp.int32, (512, 128), 1))


def _merge1_body(hist_ref, out_ref):
    C = _tile_sum(hist_ref[...].astype(jnp.float32))
    cumlt = _cumlt(C)
    cum = cumlt + C
    kf = jnp.float32(K)
    mx = (cumlt < kf) & (kf <= cum)
    flat = _flat_idx()
    b_bin = jnp.sum(jnp.where(mx, flat, 0))
    c_below = jnp.sum(jnp.where(mx, cumlt, 0.0)).astype(jnp.int32)
    key_b = lax.shift_left(b_bin, jnp.int32(16))
    vlo_bits = jnp.where(key_b < 0,
                         jnp.bitwise_and(key_b, jnp.int32(0x7FFFFFFF)),
                         jnp.bitwise_not(key_b))
    rows = lax.broadcasted_iota(jnp.int32, (4, 16), 0)
    out_ref[...] = jnp.where(rows == 0, b_bin,
                             jnp.where(rows == 1, c_below, vlo_bits))


def _merge2_body(hist_ref, sv_ref, par_ref, out_ref):
    C2 = _tile_sum(hist_ref[...].astype(jnp.float32))
    cumlt2 = _cumlt(C2)
    cum2 = cumlt2 + C2
    par = par_ref[...]
    b_bin = jnp.sum(par[0:1, 0:1])
    c_below = jnp.sum(par[1:2, 0:1]).astype(jnp.float32)
    r = jnp.float32(K) - c_below
    flat = _flat_idx()
    key = jnp.bitwise_or(lax.shift_left(b_bin, jnp.int32(16)), flat)
    u = jnp.where(key < 0, jnp.bitwise_and(key, jnp.int32(0x7FFFFFFF)),
                  jnp.bitwise_not(key))
    v = lax.bitcast_convert_type(u, jnp.float32)
    mx = (cumlt2 < r) & (r <= cum2)
    b2 = jnp.sum(jnp.where(mx, flat, 0))
    c_at = jnp.sum(jnp.where(mx, cumlt2, 0.0))
    v_thr = jnp.sum(jnp.where(mx, v, 0.0))
    s_in_bin = jnp.sum(jnp.where(flat < b2, C2 * v, 0.0))
    s_below = jnp.sum(sv_ref[...])
    total = s_below + s_in_bin + (r - c_at) * v_thr
    es = -(total / jnp.float32(K))
    out_ref[...] = jnp.broadcast_to(es, (1, 1))


_mesh = plsc.VectorSubcoreMesh(core_axis_name="c", subcore_axis_name="s")

_sc_params = pltpu.CompilerParams(needs_layout_passes=False)

_pass1 = pl.kernel(
    _pass1_body,
    out_type=jax.ShapeDtypeStruct((NW * 65536,), jnp.int32),
    mesh=_mesh,
    compiler_params=_sc_params,
    scratch_types=[
        pltpu.VMEM((65536,), jnp.int32),
        pltpu.VMEM((CHUNK,), jnp.float32),
        pltpu.VMEM((CHUNK,), jnp.float32),
        pltpu.SemaphoreType.DMA,
        pltpu.SemaphoreType.DMA,
    ],
)

_pass2 = pl.kernel(
    _pass2_body,
    out_type=(
        jax.ShapeDtypeStruct((NW * 65536,), jnp.int32),
        jax.ShapeDtypeStruct((NW, 16), jnp.float32),
    ),
    mesh=_mesh,
    compiler_params=_sc_params,
    scratch_types=[
        pltpu.VMEM((65536,), jnp.int32),
        pltpu.VMEM((CHUNK,), jnp.float32),
        pltpu.VMEM((CHUNK,), jnp.float32),
        pltpu.VMEM((16,), jnp.int32),
        pltpu.VMEM((16,), jnp.float32),
        pltpu.SemaphoreType.DMA,
        pltpu.SemaphoreType.DMA,
    ],
)

_merge1 = pl.pallas_call(
    _merge1_body,
    out_shape=jax.ShapeDtypeStruct((4, 16), jnp.int32),
)

_merge2 = pl.pallas_call(
    _merge2_body,
    out_shape=jax.ShapeDtypeStruct((1, 1), jnp.float32),
)


@jax.jit
def kernel(pnl):
    zeros = jnp.zeros((65536,), jnp.int32)
    hist1 = _pass1(pnl, zeros)
    par = _merge1(hist1.reshape(NW * 512, 128))
    hist2, sv = _pass2(pnl, par, zeros)
    out = _merge2(hist2.reshape(NW * 512, 128), sv, par)
    return out[0, 0]


# speedup vs baseline: 1.2225x; 1.0007x over previous
import jax
import jax.numpy as jnp
import numpy as np
from jax import lax
from jax.experimental import pallas as pl
from jax.experimental.pallas import tpu as pltpu
from jax.experimental.pallas import tpu_sc as plsc

N = 16777216
K = 838861
NC, NS = 2, 16
NW = NC * NS
PER_W = N // NW
CHUNK = 16384
NCHUNK = PER_W // CHUNK
UNROLL = 8
VECS = CHUNK // 16

_SIGN = np.int32(-2147483648)
_NEG1 = np.int32(-1)


def _hist_slot(x):
    xi = lax.bitcast_convert_type(x, jnp.int32)
    key = xi ^ jnp.where(xi < 0, _NEG1, _SIGN)
    hi = lax.shift_right_logical(key, jnp.int32(16))
    lo = jnp.bitwise_and(key, jnp.int32(0xFFFF))
    return hi, lo


def _pass1_body(pnl, zeros, hist_out, hist_v, buf0, buf1, sem0, sem1):
    wid = lax.axis_index("s") * NC + lax.axis_index("c")
    base = wid * PER_W
    pltpu.sync_copy(zeros, hist_v)
    bufs = (buf0, buf1)
    sems = (sem0, sem1)
    ones = jnp.ones((16,), jnp.int32)

    def copy(c, b):
        return pltpu.make_async_copy(
            pnl.at[pl.ds(base + c * CHUNK, CHUNK)], bufs[b], sems[b])

    copy(0, 0).start()

    def process(buf, carry):
        def ibody(i):
            x = buf[pl.ds(i * 16, 16)]
            hi, _ = _hist_slot(x)
            plsc.addupdate_scatter(hist_v, [hi], ones)
        plsc.parallel_loop(0, VECS, unroll=UNROLL)(ibody)
        return carry

    def outer(g, carry):
        for b in range(2):
            c = g * 2 + b
            @pl.when(c + 1 < NCHUNK)
            def _():
                copy(c + 1, 1 - b).start()
            copy(c, b).wait()
            carry = process(bufs[b], carry)
        return carry

    lax.fori_loop(0, NCHUNK // 2, outer, jnp.int32(0))
    pltpu.sync_copy(hist_v, hist_out.at[pl.ds(wid * 65536, 65536)])


def _pass2_body(pnl, par, zeros, hist_out, sv_out,
                hist_v, buf0, buf1, bvec_v, accv, sem0, sem1):
    wid = lax.axis_index("s") * NC + lax.axis_index("c")
    base = wid * PER_W
    pltpu.sync_copy(zeros, hist_v)
    pltpu.sync_copy(par.at[0], bvec_v)
    bv = bvec_v[...]
    bufs = (buf0, buf1)
    sems = (sem0, sem1)
    ones = jnp.ones((16,), jnp.int32)
    fzero = jnp.zeros((16,), jnp.float32)

    def copy(c, b):
        return pltpu.make_async_copy(
            pnl.at[pl.ds(base + c * CHUNK, CHUNK)], bufs[b], sems[b])

    copy(0, 0).start()

    def process(buf, acc):
        def ibody(i, acc):
            x = buf[pl.ds(i * 16, 16)]
            hi, lo = _hist_slot(x)
            acc = acc + jnp.where(hi < bv, x, fzero)
            plsc.addupdate_scatter(hist_v, [lo], ones, mask=hi == bv)
            return acc
        return plsc.parallel_loop(0, VECS, unroll=UNROLL, carry=acc)(ibody)

    def outer(g, acc):
        for b in range(2):
            c = g * 2 + b
            @pl.when(c + 1 < NCHUNK)
            def _():
                copy(c + 1, 1 - b).start()
            copy(c, b).wait()
            acc = process(bufs[b], acc)
        return acc

    acc = lax.fori_loop(0, NCHUNK // 2, outer, fzero)
    accv[...] = acc
    pltpu.sync_copy(accv, sv_out.at[wid])
    pltpu.sync_copy(hist_v, hist_out.at[pl.ds(wid * 65536, 65536)])


def _cumlt(C):
    rs = jnp.sum(C, axis=1, keepdims=True)
    ii = lax.broadcasted_iota(jnp.int32, (512, 512), 0)
    jj = lax.broadcasted_iota(jnp.int32, (512, 512), 1)
    lower = (jj < ii).astype(jnp.float32)
    carry = lax.dot(lower, rs, precision=lax.Precision.HIGHEST,
                    preferred_element_type=jnp.float32)
    mm = lax.broadcasted_iota(jnp.int32, (128, 128), 0)
    ll = lax.broadcasted_iota(jnp.int32, (128, 128), 1)
    upper = (mm < ll).astype(jnp.float32)
    within = lax.dot(C, upper, precision=lax.Precision.HIGHEST,
                     preferred_element_type=jnp.float32)
    return carry + within


def _tile_sum(h):
    C = h[0:512]
    for t in range(1, NW):
        C = C + h[t * 512:(t + 1) * 512]
    return C


_FLAT = None


def _flat_idx():
    return (lax.broadcasted_iota(jnp.int32, (512, 128), 0) * 128
            + lax.broadcasted_iota(jnp.int32, (512, 128), 1))


def _merge1_body(hist_ref, out_ref):
    C = _tile_sum(hist_ref[...].astype(jnp.float32))
    cumlt = _cumlt(C)
    cum = cumlt + C
    kf = jnp.float32(K)
    mx = (cumlt < kf) & (kf <= cum)
    flat = _flat_idx()
    b_bin = jnp.sum(jnp.where(mx, flat, 0))
    c_below = jnp.sum(jnp.where(mx, cumlt, 0.0)).astype(jnp.int32)
    key_b = lax.shift_left(b_bin, jnp.int32(16))
    vlo_bits = jnp.where(key_b < 0,
                         jnp.bitwise_and(key_b, jnp.int32(0x7FFFFFFF)),
                         jnp.bitwise_not(key_b))
    rows = lax.broadcasted_iota(jnp.int32, (4, 16), 0)
    out_ref[...] = jnp.where(rows == 0, b_bin,
                             jnp.where(rows == 1, c_below, vlo_bits))


def _merge2_body(hist_ref, sv_ref, par_ref, out_ref):
    C2 = _tile_sum(hist_ref[...].astype(jnp.float32))
    cumlt2 = _cumlt(C2)
    cum2 = cumlt2 + C2
    par = par_ref[...]
    b_bin = jnp.sum(par[0:1, 0:1])
    c_below = jnp.sum(par[1:2, 0:1]).astype(jnp.float32)
    r = jnp.float32(K) - c_below
    flat = _flat_idx()
    key = jnp.bitwise_or(lax.shift_left(b_bin, jnp.int32(16)), flat)
    u = jnp.where(key < 0, jnp.bitwise_and(key, jnp.int32(0x7FFFFFFF)),
                  jnp.bitwise_not(key))
    v = lax.bitcast_convert_type(u, jnp.float32)
    mx = (cumlt2 < r) & (r <= cum2)
    b2 = jnp.sum(jnp.where(mx, flat, 0))
    c_at = jnp.sum(jnp.where(mx, cumlt2, 0.0))
    v_thr = jnp.sum(jnp.where(mx, v, 0.0))
    s_in_bin = jnp.sum(jnp.where(flat < b2, C2 * v, 0.0))
    s_below = jnp.sum(sv_ref[...])
    total = s_below + s_in_bin + (r - c_at) * v_thr
    es = -(total / jnp.float32(K))
    out_ref[...] = jnp.broadcast_to(es, (1, 1))


_mesh = plsc.VectorSubcoreMesh(core_axis_name="c", subcore_axis_name="s")

_sc_params = pltpu.CompilerParams(
    needs_layout_passes=False,
    disable_bounds_checks=True,
    disable_semaphore_checks=True,
)

_pass1 = pl.kernel(
    _pass1_body,
    out_type=jax.ShapeDtypeStruct((NW * 65536,), jnp.int32),
    mesh=_mesh,
    compiler_params=_sc_params,
    scratch_types=[
        pltpu.VMEM((65536,), jnp.int32),
        pltpu.VMEM((CHUNK,), jnp.float32),
        pltpu.VMEM((CHUNK,), jnp.float32),
        pltpu.SemaphoreType.DMA,
        pltpu.SemaphoreType.DMA,
    ],
)

_pass2 = pl.kernel(
    _pass2_body,
    out_type=(
        jax.ShapeDtypeStruct((NW * 65536,), jnp.int32),
        jax.ShapeDtypeStruct((NW, 16), jnp.float32),
    ),
    mesh=_mesh,
    compiler_params=_sc_params,
    scratch_types=[
        pltpu.VMEM((65536,), jnp.int32),
        pltpu.VMEM((CHUNK,), jnp.float32),
        pltpu.VMEM((CHUNK,), jnp.float32),
        pltpu.VMEM((16,), jnp.int32),
        pltpu.VMEM((16,), jnp.float32),
        pltpu.SemaphoreType.DMA,
        pltpu.SemaphoreType.DMA,
    ],
)

_merge1 = pl.pallas_call(
    _merge1_body,
    out_shape=jax.ShapeDtypeStruct((4, 16), jnp.int32),
)

_merge2 = pl.pallas_call(
    _merge2_body,
    out_shape=jax.ShapeDtypeStruct((1, 1), jnp.float32),
)


@jax.jit
def kernel(pnl):
    zeros = jnp.zeros((65536,), jnp.int32)
    hist1 = _pass1(pnl, zeros)
    par = _merge1(hist1.reshape(NW * 512, 128))
    hist2, sv = _pass2(pnl, par, zeros)
    out = _merge2(hist2.reshape(NW * 512, 128), sv, par)
    return out[0, 0]


# speedup vs baseline: 1.2536x; 1.0254x over previous
import jax
import jax.numpy as jnp
import numpy as np
from jax import lax
from jax.experimental import pallas as pl
from jax.experimental.pallas import tpu as pltpu
from jax.experimental.pallas import tpu_sc as plsc

N = 16777216
K = 838861
NC, NS = 2, 16
NW = NC * NS
PER_W = N // NW
CHUNK = 32768
NCHUNK = PER_W // CHUNK
UNROLL = 8
VECS = CHUNK // 16
NBIN = 32768
ROWS = NBIN // 128

_SIGN = np.int32(-2147483648)
_NEG1 = np.int32(-1)


def _key32(x):
    xi = lax.bitcast_convert_type(x, jnp.int32)
    return xi ^ jnp.where(xi < 0, _NEG1, _SIGN)


def _pass1_body(pnl, zeros, hist_out, hist_v, buf0, buf1, sem0, sem1):
    wid = lax.axis_index("s") * NC + lax.axis_index("c")
    base = wid * PER_W
    pltpu.sync_copy(zeros, hist_v)
    bufs = (buf0, buf1)
    sems = (sem0, sem1)
    ones = jnp.ones((16,), jnp.int32)

    def copy(c, b):
        return pltpu.make_async_copy(
            pnl.at[pl.ds(base + c * CHUNK, CHUNK)], bufs[b], sems[b])

    copy(0, 0).start()

    def process(buf, carry):
        def ibody(i):
            x = buf[pl.ds(i * 16, 16)]
            hi = lax.shift_right_logical(_key32(x), jnp.int32(17))
            plsc.addupdate_scatter(hist_v, [hi], ones)
        plsc.parallel_loop(0, VECS, unroll=UNROLL)(ibody)
        return carry

    def outer(g, carry):
        for b in range(2):
            c = g * 2 + b
            @pl.when(c + 1 < NCHUNK)
            def _():
                copy(c + 1, 1 - b).start()
            copy(c, b).wait()
            carry = process(bufs[b], carry)
        return carry

    lax.fori_loop(0, NCHUNK // 2, outer, jnp.int32(0))
    pltpu.sync_copy(hist_v, hist_out.at[pl.ds(wid * NBIN, NBIN)])


def _pass2_body(pnl, par, zeros, hist_out, sv_out,
                hist_v, buf0, buf1, bvec_v, accv, sem0, sem1):
    wid = lax.axis_index("s") * NC + lax.axis_index("c")
    base = wid * PER_W
    pltpu.sync_copy(zeros, hist_v)
    pltpu.sync_copy(par.at[0], bvec_v)
    bv = bvec_v[...]
    bufs = (buf0, buf1)
    sems = (sem0, sem1)
    ones = jnp.ones((16,), jnp.int32)
    fzero = jnp.zeros((16,), jnp.float32)

    def copy(c, b):
        return pltpu.make_async_copy(
            pnl.at[pl.ds(base + c * CHUNK, CHUNK)], bufs[b], sems[b])

    copy(0, 0).start()

    def process(buf, acc):
        def ibody(i, acc):
            x = buf[pl.ds(i * 16, 16)]
            key = _key32(x)
            hi = lax.shift_right_logical(key, jnp.int32(17))
            lo = jnp.bitwise_and(lax.shift_right_logical(key, jnp.int32(2)),
                                 jnp.int32(0x7FFF))
            acc = acc + jnp.where(hi < bv, x, fzero)
            plsc.addupdate_scatter(hist_v, [lo], ones, mask=hi == bv)
            return acc
        return plsc.parallel_loop(0, VECS, unroll=UNROLL, carry=acc)(ibody)

    def outer(g, acc):
        for b in range(2):
            c = g * 2 + b
            @pl.when(c + 1 < NCHUNK)
            def _():
                copy(c + 1, 1 - b).start()
            copy(c, b).wait()
            acc = process(bufs[b], acc)
        return acc

    acc = lax.fori_loop(0, NCHUNK // 2, outer, fzero)
    accv[...] = acc
    pltpu.sync_copy(accv, sv_out.at[wid])
    pltpu.sync_copy(hist_v, hist_out.at[pl.ds(wid * NBIN, NBIN)])


def _cumlt(C):
    rs = jnp.sum(C, axis=1, keepdims=True)
    ii = lax.broadcasted_iota(jnp.int32, (ROWS, ROWS), 0)
    jj = lax.broadcasted_iota(jnp.int32, (ROWS, ROWS), 1)
    lower = (jj < ii).astype(jnp.float32)
    carry = lax.dot(lower, rs, precision=lax.Precision.HIGHEST,
                    preferred_element_type=jnp.float32)
    mm = lax.broadcasted_iota(jnp.int32, (128, 128), 0)
    ll = lax.broadcasted_iota(jnp.int32, (128, 128), 1)
    upper = (mm < ll).astype(jnp.float32)
    within = lax.dot(C, upper, precision=lax.Precision.HIGHEST,
                     preferred_element_type=jnp.float32)
    return carry + within


def _tile_sum(h):
    C = h[0:ROWS]
    for t in range(1, NW):
        C = C + h[t * ROWS:(t + 1) * ROWS]
    return C


def _flat_idx():
    return (lax.broadcasted_iota(jnp.int32, (ROWS, 128), 0) * 128
            + lax.broadcasted_iota(jnp.int32, (ROWS, 128), 1))


def _merge1_body(hist_ref, out_ref):
    C = _tile_sum(hist_ref[...].astype(jnp.float32))
    cumlt = _cumlt(C)
    cum = cumlt + C
    kf = jnp.float32(K)
    mx = (cumlt < kf) & (kf <= cum)
    flat = _flat_idx()
    b_bin = jnp.sum(jnp.where(mx, flat, 0))
    c_below = jnp.sum(jnp.where(mx, cumlt, 0.0)).astype(jnp.int32)
    rows = lax.broadcasted_iota(jnp.int32, (4, 16), 0)
    out_ref[...] = jnp.where(rows == 0, b_bin, c_below)


def _merge2_body(hist_ref, sv_ref, par_ref, out_ref):
    C2 = _tile_sum(hist_ref[...].astype(jnp.float32))
    cumlt2 = _cumlt(C2)
    cum2 = cumlt2 + C2
    par = par_ref[...]
    b_bin = jnp.sum(par[0:1, 0:1])
    c_below = jnp.sum(par[1:2, 0:1]).astype(jnp.float32)
    r = jnp.float32(K) - c_below
    flat = _flat_idx()
    key = jnp.bitwise_or(lax.shift_left(b_bin, jnp.int32(17)),
                         lax.shift_left(flat, jnp.int32(2)))
    u = jnp.where(key < 0, jnp.bitwise_and(key, jnp.int32(0x7FFFFFFF)),
                  jnp.bitwise_not(key))
    v = lax.bitcast_convert_type(u, jnp.float32)
    mx = (cumlt2 < r) & (r <= cum2)
    b2 = jnp.sum(jnp.where(mx, flat, 0))
    c_at = jnp.sum(jnp.where(mx, cumlt2, 0.0))
    v_thr = jnp.sum(jnp.where(mx, v, 0.0))
    s_in_bin = jnp.sum(jnp.where(flat < b2, C2 * v, 0.0))
    s_below = jnp.sum(sv_ref[...])
    total = s_below + s_in_bin + (r - c_at) * v_thr
    es = -(total / jnp.float32(K))
    out_ref[...] = jnp.broadcast_to(es, (1, 1))


_mesh = plsc.VectorSubcoreMesh(core_axis_name="c", subcore_axis_name="s")

_sc_params = pltpu.CompilerParams(needs_layout_passes=False)

_pass1 = pl.kernel(
    _pass1_body,
    out_type=jax.ShapeDtypeStruct((NW * NBIN,), jnp.int32),
    mesh=_mesh,
    compiler_params=_sc_params,
    scratch_types=[
        pltpu.VMEM((NBIN,), jnp.int32),
        pltpu.VMEM((CHUNK,), jnp.float32),
        pltpu.VMEM((CHUNK,), jnp.float32),
        pltpu.SemaphoreType.DMA,
        pltpu.SemaphoreType.DMA,
    ],
)

_pass2 = pl.kernel(
    _pass2_body,
    out_type=(
        jax.ShapeDtypeStruct((NW * NBIN,), jnp.int32),
        jax.ShapeDtypeStruct((NW, 16), jnp.float32),
    ),
    mesh=_mesh,
    compiler_params=_sc_params,
    scratch_types=[
        pltpu.VMEM((NBIN,), jnp.int32),
        pltpu.VMEM((CHUNK,), jnp.float32),
        pltpu.VMEM((CHUNK,), jnp.float32),
        pltpu.VMEM((16,), jnp.int32),
        pltpu.VMEM((16,), jnp.float32),
        pltpu.SemaphoreType.DMA,
        pltpu.SemaphoreType.DMA,
    ],
)

_merge1 = pl.pallas_call(
    _merge1_body,
    out_shape=jax.ShapeDtypeStruct((4, 16), jnp.int32),
)

_merge2 = pl.pallas_call(
    _merge2_body,
    out_shape=jax.ShapeDtypeStruct((1, 1), jnp.float32),
)


@jax.jit
def kernel(pnl):
    zeros = jnp.zeros((NBIN,), jnp.int32)
    hist1 = _pass1(pnl, zeros)
    par = _merge1(hist1.reshape(NW * ROWS, 128))
    hist2, sv = _pass2(pnl, par, zeros)
    out = _merge2(hist2.reshape(NW * ROWS, 128), sv, par)
    return out[0, 0]
